# Initial kernel scaffold; baseline (speedup 1.0000x reference)
import jax
import jax.numpy as jnp
from jax.experimental import pallas as pl


def kernel(h, edge_index, coord, edge_attr, W_e1, b_e1, W_e2, b_e2, W_n1, b_n1, W_n2, b_n2, W_c1, b_c1, W_c2, b_c2):
    raise NotImplementedError("write your pallas kernel here")


# speedup vs baseline: 3.3066x; 3.3066x over previous
import functools

import jax
import jax.numpy as jnp
from jax import lax
from jax.experimental import pallas as pl
from jax.experimental.pallas import tpu as pltpu
from jax.experimental.pallas import tpu_sc as plsc

N = 10000
E = 320000
D = 128
DE = 16
H = 128

NC = 2
NS = 16
NW = NC * NS
EPW = E // NW
GB = 80
SB = 160
NP = 10240
HALF = NP // 2
ACC_R = 5376
NPT = ACC_R // NS
DPT = HALF // NS
EPT = E // NS

def _pq_body(h_ref, wa_ref, wb_ref, be1_ref, p_ref, q_ref):
    hb = h_ref[...]
    p_ref[...] = hb @ wa_ref[...] + be1_ref[...]
    q_ref[...] = hb @ wb_ref[...]


def _pq_call(h, wa, wb, be1):
    nb = 1000
    return pl.pallas_call(
        _pq_body,
        grid=(N // nb,),
        in_specs=[
            pl.BlockSpec((nb, D), lambda i: (i, 0)),
            pl.BlockSpec((D, H), lambda i: (0, 0)),
            pl.BlockSpec((D, H), lambda i: (0, 0)),
            pl.BlockSpec((1, H), lambda i: (0, 0)),
        ],
        out_specs=[
            pl.BlockSpec((nb, H), lambda i: (i, 0)),
            pl.BlockSpec((nb, H), lambda i: (i, 0)),
        ],
        out_shape=[
            jax.ShapeDtypeStruct((N, H), jnp.float32),
            jax.ShapeDtypeStruct((N, H), jnp.float32),
        ],
    )(h, wa, wb, be1)


def _gather_body(row_hbm, col_hbm, p_hbm, q_hbm, c4_hbm,
                 pg_hbm, qg_hbm, d4_hbm,
                 idxr, idxc, bufp, bufq, ctab, d4buf,
                 sem0, sem1):
    wid = lax.axis_index("s") * NC + lax.axis_index("c")
    base = wid * EPW
    pltpu.sync_copy(c4_hbm, ctab)
    lane4 = lax.iota(jnp.int32, 16) * 4

    def chunk(i, _):
        off = base + i * GB
        pltpu.sync_copy(row_hbm.at[pl.ds(off, GB)], idxr)
        pltpu.sync_copy(col_hbm.at[pl.ds(off, GB)], idxc)
        cp0 = pltpu.async_copy(p_hbm.at[idxr], bufp, sem0)
        cp1 = pltpu.async_copy(q_hbm.at[idxc], bufq, sem1)

        def jloop(j, _):
            r16 = idxr[pl.ds(j * 16, 16)]
            c16 = idxc[pl.ds(j * 16, 16)]
            rad = jnp.zeros((16,), jnp.float32)
            r4 = r16 * 4
            c4i = c16 * 4
            for c in range(3):
                d = (plsc.load_gather(ctab, [r4 + c])
                     - plsc.load_gather(ctab, [c4i + c]))
                plsc.store_scatter(d4buf, [lane4 + (j * 64 + c)], d)
                rad = rad + d * d
            plsc.store_scatter(d4buf, [lane4 + (j * 64 + 3)], rad)
            return _
        lax.fori_loop(0, GB // 16, jloop, None)

        cp0.wait()
        cp1.wait()
        pltpu.sync_copy(bufp, pg_hbm.at[pl.ds(off, GB)])
        pltpu.sync_copy(bufq, qg_hbm.at[pl.ds(off, GB)])
        pltpu.sync_copy(d4buf, d4_hbm.at[pl.ds(off * 4, GB * 4)])
        return _

    lax.fori_loop(0, EPW // GB, chunk, None)


def _gather_call(row, col, p, q, c4):
    mesh = plsc.VectorSubcoreMesh(core_axis_name="c", subcore_axis_name="s")
    f = pl.kernel(
        _gather_body,
        out_type=[
            jax.ShapeDtypeStruct((E, H), jnp.float32),
            jax.ShapeDtypeStruct((E, H), jnp.float32),
            jax.ShapeDtypeStruct((E * 4,), jnp.float32),
        ],
        mesh=mesh,
        compiler_params=pltpu.CompilerParams(needs_layout_passes=False),
        scratch_types=[
            pltpu.VMEM((GB,), jnp.int32),
            pltpu.VMEM((GB,), jnp.int32),
            pltpu.VMEM((GB, H), jnp.float32),
            pltpu.VMEM((GB, H), jnp.float32),
            pltpu.VMEM((N * 4,), jnp.float32),
            pltpu.VMEM((GB * 4,), jnp.float32),
            pltpu.SemaphoreType.DMA,
            pltpu.SemaphoreType.DMA,
        ],
    )
    return f(row, col, p, q, c4)


def _edge_body(pg_ref, qg_ref, d4_ref, ea_ref,
               wr_ref, wa2_ref, we2_ref, be2_ref,
               wc1_ref, bc1_ref, wc2_ref, bc2_ref,
               m_ref, t4_ref):
    d4 = d4_ref[...]
    radial = d4[:, 3:4]
    pre = (pg_ref[...] + qg_ref[...] + radial * wr_ref[...]
           + ea_ref[...] @ wa2_ref[...])
    m1 = jnp.maximum(pre, 0.0)
    m = jnp.maximum(m1 @ we2_ref[...] + be2_ref[...], 0.0)
    m_ref[...] = m
    cw = (jnp.maximum(m @ wc1_ref[...] + bc1_ref[...], 0.0) @ wc2_ref[...]
          + bc2_ref[...])
    lane = lax.broadcasted_iota(jnp.int32, d4.shape, 1)
    t4_ref[...] = jnp.where(lane < 3, d4 * cw, 1.0)


def _edge_call(pg, qg, d4, ea, wr, wa2, we2, be2, wc1, bc1, wc2, bc2):
    eb = 2000
    full = lambda r, c: pl.BlockSpec((r, c), lambda i: (0, 0))
    return pl.pallas_call(
        _edge_body,
        grid=(E // eb,),
        in_specs=[
            pl.BlockSpec((eb, H), lambda i: (i, 0)),
            pl.BlockSpec((eb, H), lambda i: (i, 0)),
            pl.BlockSpec((eb, 4), lambda i: (i, 0)),
            pl.BlockSpec((eb, DE), lambda i: (i, 0)),
            full(1, H), full(DE, H), full(H, H), full(1, H),
            full(H, H), full(1, H), full(H, 1), full(1, 1),
        ],
        out_specs=[
            pl.BlockSpec((eb, H), lambda i: (i, 0)),
            pl.BlockSpec((eb, 4), lambda i: (i, 0)),
        ],
        out_shape=[
            jax.ShapeDtypeStruct((E, H), jnp.float32),
            jax.ShapeDtypeStruct((E, 4), jnp.float32),
        ],
    )(pg, qg, d4, ea, wr, wa2, we2, be2, wc1, bc1, wc2, bc2)


def _scatter_body(row_hbm, m_hbm, t4f_hbm, z2d_hbm,
                  agg_hbm, t4p_hbm,
                  idxb, idxs, mbuf, t4buf, acc4, agg_sp):
    cid = lax.axis_index("c")
    sid = lax.axis_index("s")
    base = sid * EPT

    @pl.when(cid == 0)
    def _z4():
        def z4(t, _):
            acc4[pl.ds(t * 16, 16)] = jnp.zeros((16,), jnp.float32)
            return _
        lax.fori_loop(0, (4 * N) // 16, z4, None)

    pltpu.sync_copy(z2d_hbm, agg_sp.at[pl.ds(sid * NPT, NPT)])
    plsc.subcore_barrier()

    lane16 = lax.iota(jnp.int32, 16)
    ones16 = jnp.ones((16,), jnp.float32)
    lo = cid * HALF

    def chunk(i, _):
        off = base + i * SB
        pltpu.sync_copy(row_hbm.at[pl.ds(off, SB)], idxb)
        pltpu.sync_copy(m_hbm.at[pl.ds(off, SB)], mbuf)

        def kloop(k, _):
            v = idxb[pl.ds(k * 16, 16)] - lo
            ok = jnp.logical_and(v >= 0, v < HALF)
            idxs[pl.ds(k * 16, 16)] = jnp.where(ok, v, HALF)
            return _
        lax.fori_loop(0, SB // 16, kloop, None)

        pltpu.sync_copy(mbuf, agg_sp.at[idxs], add=True)

        @pl.when(cid == 0)
        def _t4():
            pltpu.sync_copy(t4f_hbm.at[pl.ds(off * 4, SB * 4)], t4buf)

            def jloop(j, _):
                nid = idxb[pl.ds(j * 16, 16)]
                tgt = nid * 4
                src = lane16 * 4 + j * 64
                for c in range(3):
                    val = plsc.load_gather(t4buf, [src + c])
                    plsc.addupdate_scatter(acc4, [tgt + c], val)
                plsc.addupdate_scatter(acc4, [tgt + 3], ones16)
                return _
            lax.fori_loop(0, SB // 16, jloop, None)
        return _

    lax.fori_loop(0, EPT // SB, chunk, None)
    plsc.subcore_barrier()

    pltpu.sync_copy(agg_sp.at[pl.ds(sid * DPT, DPT)],
                    agg_hbm.at[pl.ds(cid * HALF + sid * DPT, DPT)])

    @pl.when(cid == 0)
    def _d4():
        def dump4(k, _):
            pltpu.sync_copy(acc4.at[pl.ds(k * 5000, 5000)],
                            t4p_hbm.at[pl.ds(sid * 4 * N + k * 5000, 5000)])
            return _
        lax.fori_loop(0, 8, dump4, None)


def _scatter_call(row, m, t4f, z2d):
    mesh = plsc.VectorSubcoreMesh(core_axis_name="c", subcore_axis_name="s")
    f = pl.kernel(
        _scatter_body,
        out_type=[
            jax.ShapeDtypeStruct((NP, H), jnp.float32),
            jax.ShapeDtypeStruct((NS * 4 * N,), jnp.float32),
        ],
        mesh=mesh,
        compiler_params=pltpu.CompilerParams(needs_layout_passes=False),
        scratch_types=[
            pltpu.VMEM((SB,), jnp.int32),
            pltpu.VMEM((SB,), jnp.int32),
            pltpu.VMEM((SB, H), jnp.float32),
            pltpu.VMEM((SB * 4,), jnp.float32),
            pltpu.VMEM((4 * N,), jnp.float32),
            pltpu.VMEM_SHARED((ACC_R, H), jnp.float32),
        ],
    )
    return f(row, m, t4f, z2d)


def _node_body(h_ref, agg_ref, trp_ref, coord_ref,
               wn1h_ref, wn1a_ref, bn1_ref, wn2_ref, bn2_ref,
               hout_ref, cout_ref):
    t1 = jnp.maximum(h_ref[...] @ wn1h_ref[...] + agg_ref[...] @ wn1a_ref[...]
                     + bn1_ref[...], 0.0)
    hout_ref[...] = t1 @ wn2_ref[...] + bn2_ref[...]
    tr = jnp.sum(trp_ref[...], axis=0)
    cnt = jnp.maximum(tr[:, 3:4], 1.0)
    cout_ref[...] = coord_ref[...] + tr[:, 0:3] / cnt


def _node_call(h, agg, trp, coord, wn1h, wn1a, bn1, wn2, bn2):
    nb = 1000
    full = lambda r, c: pl.BlockSpec((r, c), lambda i: (0, 0))
    return pl.pallas_call(
        _node_body,
        grid=(N // nb,),
        in_specs=[
            pl.BlockSpec((nb, D), lambda i: (i, 0)),
            pl.BlockSpec((nb, H), lambda i: (i, 0)),
            pl.BlockSpec((NS, nb, 4), lambda i: (0, i, 0)),
            pl.BlockSpec((nb, 3), lambda i: (i, 0)),
            full(D, H), full(H, H), full(1, H), full(H, H), full(1, H),
        ],
        out_specs=[
            pl.BlockSpec((nb, H), lambda i: (i, 0)),
            pl.BlockSpec((nb, 3), lambda i: (i, 0)),
        ],
        out_shape=[
            jax.ShapeDtypeStruct((N, H), jnp.float32),
            jax.ShapeDtypeStruct((N, 3), jnp.float32),
        ],
    )(h, agg, trp, coord, wn1h, wn1a, bn1, wn2, bn2)


@jax.jit
def kernel(h, edge_index, coord, edge_attr,
           W_e1, b_e1, W_e2, b_e2,
           W_n1, b_n1, W_n2, b_n2,
           W_c1, b_c1, W_c2, b_c2):
    row = edge_index[0]
    col = edge_index[1]

    wa = W_e1[:D]
    wb = W_e1[D:2 * D]
    wr = W_e1[2 * D:2 * D + 1]
    wa2 = W_e1[2 * D + 1:]

    p, q = _pq_call(h, wa, wb, b_e1.reshape(1, H))

    c4 = jnp.pad(coord, ((0, 0), (0, 1))).reshape(N * 4)
    pg, qg, d4f = _gather_call(row, col, p, q, c4)

    m_ij, t4 = _edge_call(
        pg, qg, d4f.reshape(E, 4), edge_attr,
        wr, wa2, W_e2, b_e2.reshape(1, H),
        W_c1, b_c1.reshape(1, H), W_c2, b_c2.reshape(1, 1))

    z2d = jnp.zeros((NPT, H), jnp.float32)
    agg, t4p = _scatter_call(row, m_ij, t4.reshape(E * 4), z2d)

    h_out, coord_out = _node_call(
        h, agg, t4p.reshape(NS, N, 4), coord,
        W_n1[:D], W_n1[D:], b_n1.reshape(1, H),
        W_n2, b_n2.reshape(1, H))

    return (h_out, coord_out, m_ij)


# speedup vs baseline: 3.3774x; 1.0214x over previous
import functools

import jax
import jax.numpy as jnp
from jax import lax
from jax.experimental import pallas as pl
from jax.experimental.pallas import tpu as pltpu
from jax.experimental.pallas import tpu_sc as plsc

N = 10000
E = 320000
D = 128
DE = 16
H = 128

NC = 2
NS = 16
NW = NC * NS
EPW = E // NW
GB = 80
SB = 80
NP = 10240
HALF = NP // 2
ACC_R = 5376
NPT = ACC_R // NS
DPT = HALF // NS
EPT = E // NS

def _pq_body(h_ref, wa_ref, wb_ref, be1_ref, p_ref, q_ref):
    hb = h_ref[...]
    p_ref[...] = hb @ wa_ref[...] + be1_ref[...]
    q_ref[...] = hb @ wb_ref[...]


def _pq_call(h, wa, wb, be1):
    nb = 1000
    return pl.pallas_call(
        _pq_body,
        grid=(N // nb,),
        in_specs=[
            pl.BlockSpec((nb, D), lambda i: (i, 0)),
            pl.BlockSpec((D, H), lambda i: (0, 0)),
            pl.BlockSpec((D, H), lambda i: (0, 0)),
            pl.BlockSpec((1, H), lambda i: (0, 0)),
        ],
        out_specs=[
            pl.BlockSpec((nb, H), lambda i: (i, 0)),
            pl.BlockSpec((nb, H), lambda i: (i, 0)),
        ],
        out_shape=[
            jax.ShapeDtypeStruct((N, H), jnp.float32),
            jax.ShapeDtypeStruct((N, H), jnp.float32),
        ],
    )(h, wa, wb, be1)


def _gather_body(row_hbm, col_hbm, p_hbm, q_hbm, c4_hbm,
                 s_hbm, d4_hbm,
                 idxr, idxc, bufp0, bufp1, bufq0, bufq1, ctab, d4b0, d4b1,
                 semg, semw0, semw1, semd0, semd1):
    wid = lax.axis_index("s") * NC + lax.axis_index("c")
    base = wid * EPW
    bufp = (bufp0, bufp1)
    bufq = (bufq0, bufq1)
    d4b = (d4b0, d4b1)
    semw = (semw0, semw1)
    semd = (semd0, semd1)
    NCH = EPW // GB

    pltpu.sync_copy(row_hbm.at[pl.ds(base, EPW)], idxr)
    pltpu.sync_copy(col_hbm.at[pl.ds(base, EPW)], idxc)
    pltpu.sync_copy(c4_hbm, ctab)
    lane4 = lax.iota(jnp.int32, 16) * 4

    def do_chunk(i, b, drain):
        off = base + i * GB
        ioff = i * GB
        if drain:
            pltpu.make_async_copy(bufp[b], s_hbm.at[pl.ds(0, GB)], semw[b]).wait()
            pltpu.make_async_copy(d4b[b], d4_hbm.at[pl.ds(0, GB * 4)], semd[b]).wait()
        cpp = pltpu.async_copy(p_hbm.at[idxr.at[pl.ds(ioff, GB)]], bufp[b], semg)
        cpq = pltpu.async_copy(q_hbm.at[idxc.at[pl.ds(ioff, GB)]], bufq[b], semg)

        def jloop(j, _):
            r16 = idxr[pl.ds(ioff + j * 16, 16)]
            c16 = idxc[pl.ds(ioff + j * 16, 16)]
            rad = jnp.zeros((16,), jnp.float32)
            r4 = r16 * 4
            c4i = c16 * 4
            for c in range(3):
                d = (plsc.load_gather(ctab, [r4 + c])
                     - plsc.load_gather(ctab, [c4i + c]))
                plsc.store_scatter(d4b[b], [lane4 + (j * 64 + c)], d)
                rad = rad + d * d
            plsc.store_scatter(d4b[b], [lane4 + (j * 64 + 3)], rad)
            return _
        lax.fori_loop(0, GB // 16, jloop, None)

        cpp.wait()
        cpq.wait()

        def aloop(t, _):
            j = t // 8
            k = (t % 8) * 16
            bufp[b][j, pl.ds(k, 16)] = (bufp[b][j, pl.ds(k, 16)]
                                        + bufq[b][j, pl.ds(k, 16)])
            return _
        lax.fori_loop(0, GB * 8, aloop, None)

        pltpu.async_copy(bufp[b], s_hbm.at[pl.ds(off, GB)], semw[b])
        pltpu.async_copy(d4b[b], d4_hbm.at[pl.ds(off * 4, GB * 4)], semd[b])

    do_chunk(0, 0, False)
    do_chunk(1, 1, False)

    def pair(t, _):
        do_chunk(2 * t + 2, 0, True)
        do_chunk(2 * t + 3, 1, True)
        return _
    lax.fori_loop(0, (NCH - 3) // 2, pair, None)
    do_chunk(NCH - 1, 0, True)

    pltpu.make_async_copy(bufp[0], s_hbm.at[pl.ds(0, GB)], semw0).wait()
    pltpu.make_async_copy(d4b0, d4_hbm.at[pl.ds(0, GB * 4)], semd0).wait()
    pltpu.make_async_copy(bufp[1], s_hbm.at[pl.ds(0, GB)], semw1).wait()
    pltpu.make_async_copy(d4b1, d4_hbm.at[pl.ds(0, GB * 4)], semd1).wait()


def _gather_call(row, col, p, q, c4):
    mesh = plsc.VectorSubcoreMesh(core_axis_name="c", subcore_axis_name="s")
    f = pl.kernel(
        _gather_body,
        out_type=[
            jax.ShapeDtypeStruct((E, H), jnp.float32),
            jax.ShapeDtypeStruct((E * 4,), jnp.float32),
        ],
        mesh=mesh,
        compiler_params=pltpu.CompilerParams(needs_layout_passes=False),
        scratch_types=[
            pltpu.VMEM((EPW,), jnp.int32),
            pltpu.VMEM((EPW,), jnp.int32),
            pltpu.VMEM((GB, H), jnp.float32),
            pltpu.VMEM((GB, H), jnp.float32),
            pltpu.VMEM((GB, H), jnp.float32),
            pltpu.VMEM((GB, H), jnp.float32),
            pltpu.VMEM((N * 4,), jnp.float32),
            pltpu.VMEM((GB * 4,), jnp.float32),
            pltpu.VMEM((GB * 4,), jnp.float32),
            pltpu.SemaphoreType.DMA,
            pltpu.SemaphoreType.DMA,
            pltpu.SemaphoreType.DMA,
            pltpu.SemaphoreType.DMA,
            pltpu.SemaphoreType.DMA,
        ],
    )
    return f(row, col, p, q, c4)


def _edge_body(s_ref, d4_ref, ea_ref,
               wr_ref, wa2_ref, we2_ref, be2_ref,
               wc1_ref, bc1_ref, wc2_ref, bc2_ref,
               m_ref, t4_ref):
    d4 = d4_ref[...]
    radial = d4[:, 3:4]
    pre = (s_ref[...] + radial * wr_ref[...]
           + ea_ref[...] @ wa2_ref[...])
    m1 = jnp.maximum(pre, 0.0)
    m = jnp.maximum(m1 @ we2_ref[...] + be2_ref[...], 0.0)
    m_ref[...] = m
    cw = (jnp.maximum(m @ wc1_ref[...] + bc1_ref[...], 0.0) @ wc2_ref[...]
          + bc2_ref[...])
    lane = lax.broadcasted_iota(jnp.int32, d4.shape, 1)
    t4_ref[...] = jnp.where(lane < 3, d4 * cw, 1.0)


def _edge_call(sarr, d4, ea, wr, wa2, we2, be2, wc1, bc1, wc2, bc2):
    eb = 2000
    full = lambda r, c: pl.BlockSpec((r, c), lambda i: (0, 0))
    return pl.pallas_call(
        _edge_body,
        grid=(E // eb,),
        in_specs=[
            pl.BlockSpec((eb, H), lambda i: (i, 0)),
            pl.BlockSpec((eb, 4), lambda i: (i, 0)),
            pl.BlockSpec((eb, DE), lambda i: (i, 0)),
            full(1, H), full(DE, H), full(H, H), full(1, H),
            full(H, H), full(1, H), full(H, 1), full(1, 1),
        ],
        out_specs=[
            pl.BlockSpec((eb, H), lambda i: (i, 0)),
            pl.BlockSpec((eb, 4), lambda i: (i, 0)),
        ],
        out_shape=[
            jax.ShapeDtypeStruct((E, H), jnp.float32),
            jax.ShapeDtypeStruct((E, 4), jnp.float32),
        ],
    )(sarr, d4, ea, wr, wa2, we2, be2, wc1, bc1, wc2, bc2)


def _scatter_body(row_hbm, m_hbm, t4f_hbm, z2d_hbm,
                  agg_hbm, t4p_hbm,
                  idxa, idxs0, idxs1, mbuf0, mbuf1, t4b0, t4b1, acc4, agg_sp,
                  semm0, semm1, sema0, sema1, semt0, semt1):
    cid = lax.axis_index("c")
    sid = lax.axis_index("s")
    base = sid * EPT
    idxs = (idxs0, idxs1)
    mbuf = (mbuf0, mbuf1)
    t4b = (t4b0, t4b1)
    semm = (semm0, semm1)
    sema = (sema0, sema1)
    semt = (semt0, semt1)
    NCH = EPT // SB

    @pl.when(cid == 0)
    def _z4():
        def z4(t, _):
            acc4[pl.ds(t * 16, 16)] = jnp.zeros((16,), jnp.float32)
            return _
        lax.fori_loop(0, (4 * N) // 16, z4, None)

    pltpu.sync_copy(z2d_hbm, agg_sp.at[pl.ds(sid * NPT, NPT)])
    pltpu.sync_copy(row_hbm.at[pl.ds(base, EPT)], idxa)
    plsc.subcore_barrier()

    lane16 = lax.iota(jnp.int32, 16)
    ones16 = jnp.ones((16,), jnp.float32)
    lo = cid * HALF

    def do_chunk(i, b, drain, prefetch_t4):
        off = base + i * SB
        ioff = i * SB
        if drain:
            pltpu.make_async_copy(mbuf[b], agg_sp.at[idxs[b]], sema[b]).wait()
        pltpu.async_copy(m_hbm.at[pl.ds(off, SB)], mbuf[b], semm[b])

        def kloop(k, _):
            v = idxa[pl.ds(ioff + k * 16, 16)] - lo
            ok = jnp.logical_and(v >= 0, v < HALF)
            idxs[b][pl.ds(k * 16, 16)] = jnp.where(ok, v, HALF)
            return _
        lax.fori_loop(0, SB // 16, kloop, None)

        @pl.when(cid == 0)
        def _t4pre():
            if prefetch_t4:
                pltpu.async_copy(
                    t4f_hbm.at[pl.ds((off + SB) * 4, SB * 4)],
                    t4b[1 - b], semt[1 - b])

        pltpu.make_async_copy(m_hbm.at[pl.ds(0, SB)], mbuf[b], semm[b]).wait()
        pltpu.async_copy(mbuf[b], agg_sp.at[idxs[b]], sema[b], add=True)

        @pl.when(cid == 0)
        def _t4():
            pltpu.make_async_copy(t4f_hbm.at[pl.ds(0, SB * 4)],
                                  t4b[b], semt[b]).wait()

            def jloop(j, _):
                nid = idxa[pl.ds(ioff + j * 16, 16)]
                tgt = nid * 4
                src = lane16 * 4 + j * 64
                for c in range(3):
                    val = plsc.load_gather(t4b[b], [src + c])
                    plsc.addupdate_scatter(acc4, [tgt + c], val)
                plsc.addupdate_scatter(acc4, [tgt + 3], ones16)
                return _
            lax.fori_loop(0, SB // 16, jloop, None)

    @pl.when(cid == 0)
    def _t4prime():
        pltpu.async_copy(t4f_hbm.at[pl.ds(base * 4, SB * 4)], t4b[0], semt[0])

    do_chunk(0, 0, False, True)
    do_chunk(1, 1, False, True)

    def pair(t, _):
        do_chunk(2 * t + 2, 0, True, True)
        do_chunk(2 * t + 3, 1, True, True)
        return _
    lax.fori_loop(0, (NCH - 4) // 2, pair, None)
    do_chunk(NCH - 2, 0, True, True)
    do_chunk(NCH - 1, 1, True, False)

    pltpu.make_async_copy(mbuf[0], agg_sp.at[idxs[0]], sema[0]).wait()
    pltpu.make_async_copy(mbuf[1], agg_sp.at[idxs[1]], sema[1]).wait()
    plsc.subcore_barrier()

    pltpu.sync_copy(agg_sp.at[pl.ds(sid * DPT, DPT)],
                    agg_hbm.at[pl.ds(cid * HALF + sid * DPT, DPT)])

    @pl.when(cid == 0)
    def _d4():
        def dump4(k, _):
            pltpu.sync_copy(acc4.at[pl.ds(k * 5000, 5000)],
                            t4p_hbm.at[pl.ds(sid * 4 * N + k * 5000, 5000)])
            return _
        lax.fori_loop(0, 8, dump4, None)


def _scatter_call(row, m, t4f, z2d):
    mesh = plsc.VectorSubcoreMesh(core_axis_name="c", subcore_axis_name="s")
    f = pl.kernel(
        _scatter_body,
        out_type=[
            jax.ShapeDtypeStruct((NP, H), jnp.float32),
            jax.ShapeDtypeStruct((NS * 4 * N,), jnp.float32),
        ],
        mesh=mesh,
        compiler_params=pltpu.CompilerParams(needs_layout_passes=False),
        scratch_types=[
            pltpu.VMEM((EPT,), jnp.int32),
            pltpu.VMEM((SB,), jnp.int32),
            pltpu.VMEM((SB,), jnp.int32),
            pltpu.VMEM((SB, H), jnp.float32),
            pltpu.VMEM((SB, H), jnp.float32),
            pltpu.VMEM((SB * 4,), jnp.float32),
            pltpu.VMEM((SB * 4,), jnp.float32),
            pltpu.VMEM((4 * N,), jnp.float32),
            pltpu.VMEM_SHARED((ACC_R, H), jnp.float32),
            pltpu.SemaphoreType.DMA,
            pltpu.SemaphoreType.DMA,
            pltpu.SemaphoreType.DMA,
            pltpu.SemaphoreType.DMA,
            pltpu.SemaphoreType.DMA,
            pltpu.SemaphoreType.DMA,
        ],
    )
    return f(row, m, t4f, z2d)


def _node_body(h_ref, agg_ref, trp_ref, coord_ref,
               wn1h_ref, wn1a_ref, bn1_ref, wn2_ref, bn2_ref,
               hout_ref, cout_ref):
    t1 = jnp.maximum(h_ref[...] @ wn1h_ref[...] + agg_ref[...] @ wn1a_ref[...]
                     + bn1_ref[...], 0.0)
    hout_ref[...] = t1 @ wn2_ref[...] + bn2_ref[...]
    tr = jnp.sum(trp_ref[...], axis=0)
    cnt = jnp.maximum(tr[:, 3:4], 1.0)
    cout_ref[...] = coord_ref[...] + tr[:, 0:3] / cnt


def _node_call(h, agg, trp, coord, wn1h, wn1a, bn1, wn2, bn2):
    nb = 1000
    full = lambda r, c: pl.BlockSpec((r, c), lambda i: (0, 0))
    return pl.pallas_call(
        _node_body,
        grid=(N // nb,),
        in_specs=[
            pl.BlockSpec((nb, D), lambda i: (i, 0)),
            pl.BlockSpec((nb, H), lambda i: (i, 0)),
            pl.BlockSpec((NS, nb, 4), lambda i: (0, i, 0)),
            pl.BlockSpec((nb, 3), lambda i: (i, 0)),
            full(D, H), full(H, H), full(1, H), full(H, H), full(1, H),
        ],
        out_specs=[
            pl.BlockSpec((nb, H), lambda i: (i, 0)),
            pl.BlockSpec((nb, 3), lambda i: (i, 0)),
        ],
        out_shape=[
            jax.ShapeDtypeStruct((N, H), jnp.float32),
            jax.ShapeDtypeStruct((N, 3), jnp.float32),
        ],
    )(h, agg, trp, coord, wn1h, wn1a, bn1, wn2, bn2)


@jax.jit
def kernel(h, edge_index, coord, edge_attr,
           W_e1, b_e1, W_e2, b_e2,
           W_n1, b_n1, W_n2, b_n2,
           W_c1, b_c1, W_c2, b_c2):
    row = edge_index[0]
    col = edge_index[1]

    wa = W_e1[:D]
    wb = W_e1[D:2 * D]
    wr = W_e1[2 * D:2 * D + 1]
    wa2 = W_e1[2 * D + 1:]

    p, q = _pq_call(h, wa, wb, b_e1.reshape(1, H))

    c4 = jnp.pad(coord, ((0, 0), (0, 1))).reshape(N * 4)
    sarr, d4f = _gather_call(row, col, p, q, c4)

    m_ij, t4 = _edge_call(
        sarr, d4f.reshape(E, 4), edge_attr,
        wr, wa2, W_e2, b_e2.reshape(1, H),
        W_c1, b_c1.reshape(1, H), W_c2, b_c2.reshape(1, 1))

    z2d = jnp.zeros((NPT, H), jnp.float32)
    agg, t4p = _scatter_call(row, m_ij, t4.reshape(E * 4), z2d)

    h_out, coord_out = _node_call(
        h, agg, t4p.reshape(NS, N, 4), coord,
        W_n1[:D], W_n1[D:], b_n1.reshape(1, H),
        W_n2, b_n2.reshape(1, H))

    return (h_out, coord_out, m_ij)
